# baseline (device time: 186146 ns/iter reference)
import jax
import jax.numpy as jnp
from jax import lax
from jax.experimental import pallas as pl
from jax.experimental.pallas import tpu as pltpu

N_DEV = 4

FROM_L = 0
FROM_R = 1
OPP = 2


def kernel(x, router_W, route_idx, expert_W, shared_W):
    n_tok, d_model = x.shape
    e_per, _, d_h = expert_W.shape
    e_half = e_per // 2
    e_q = e_half // 2

    xb = x.astype(jnp.bfloat16)
    rwb = router_W.astype(jnp.bfloat16)
    swb = shared_W.astype(jnp.bfloat16)

    def body(x_ref, rw_ref, idx_ref, ew_ref, sw_ref, out_ref,
             gather_ref, stage_ref, own_ref, p_ref, send_sems, recv_sems,
             load_sems):
        my = lax.axis_index("i")
        left = lax.rem(my + N_DEV - 1, N_DEV)
        right = lax.rem(my + 1, N_DEV)

        def rdma(src, dst, si, ri, dev):
            return pltpu.make_async_remote_copy(
                src_ref=src, dst_ref=dst,
                send_sem=send_sems.at[si], recv_sem=recv_sems.at[ri],
                device_id=(dev,), device_id_type=pl.DeviceIdType.MESH,
            )

        def stage_round(r):
            ld = pltpu.make_async_copy(
                ew_ref.at[pl.ds(r * e_q, e_q)], stage_ref,
                load_sems.at[r % 2])
            ld.start()
            ld.wait()
            for k in range(e_q):
                own_ref[r * e_q + k] = stage_ref[k].astype(jnp.bfloat16)

        stage_round(0)

        barrier_sem = pltpu.get_barrier_semaphore()
        for nbr in (left, right):
            pl.semaphore_signal(
                barrier_sem, inc=1,
                device_id=(nbr,), device_id_type=pl.DeviceIdType.MESH,
            )
        pl.semaphore_wait(barrier_sem, 2)
        stage_round(1)

        lo = pl.ds(0, e_half)
        hi = pl.ds(e_half, e_half)
        s1_lo_r = rdma(own_ref.at[lo], gather_ref.at[FROM_L, lo], 0, 0, right)
        s1_lo_l = rdma(own_ref.at[lo], gather_ref.at[FROM_R, lo], 1, 1, left)
        s1_hi_r = rdma(own_ref.at[hi], gather_ref.at[FROM_L, hi], 2, 2, right)
        s1_hi_l = rdma(own_ref.at[hi], gather_ref.at[FROM_R, hi], 3, 3, left)
        s1_lo_r.start()
        s1_lo_l.start()

        stage_round(2)
        stage_round(3)

        idx = idx_ref[...]
        scores = jnp.dot(x_ref[...], rw_ref[...],
                         preferred_element_type=jnp.float32)
        s_max = jnp.max(scores, axis=-1, keepdims=True)
        e_s = jnp.exp(scores - s_max)
        probs = e_s / jnp.sum(e_s, axis=-1, keepdims=True)
        eids = lax.broadcasted_iota(jnp.int32, scores.shape, 1)
        p_ref[...] = jnp.sum(jnp.where(eids == idx, probs, 0.0),
                             axis=-1, keepdims=True)

        s1_lo_r.wait_send()
        s1_hi_r.start()
        s1_lo_l.wait_send()
        s1_hi_l.start()

        TB = 512
        NB = n_tok // TB

        sw = sw_ref[...]

        def shared_blk(b, c):
            sl = pl.ds(b * TB, TB)
            out_ref[sl, :] = jnp.dot(
                x_ref[sl, :], sw, preferred_element_type=jnp.float32
            ).astype(jnp.bfloat16)
            return c

        lax.fori_loop(0, NB, shared_blk, 0)

        def add_chunk(origin, w_ref, j0=0, nj=e_per):
            def blk(b, c):
                sl = pl.ds(b * TB, TB)
                x_blk = x_ref[sl, :]
                idx_blk = idx_ref[sl, :]
                p_blk = p_ref[sl, :]
                acc = out_ref[sl, :].astype(jnp.float32)
                for j in range(j0, j0 + nj):
                    e_glob = origin * e_per + j
                    coeff = jnp.where(idx_blk == e_glob, p_blk,
                                      0.0).astype(jnp.bfloat16)
                    acc = acc + jnp.dot(x_blk * coeff, w_ref[j],
                                        preferred_element_type=jnp.float32)
                out_ref[sl, :] = acc.astype(jnp.bfloat16)
                return c

            lax.fori_loop(0, NB, blk, 0)

        add_chunk(my, own_ref)

        q = [pl.ds(k * e_q, e_q) for k in range(4)]
        f2l_q0 = rdma(gather_ref.at[FROM_R, q[0]], gather_ref.at[OPP, q[0]],
                      4, 4, left)
        f2l_q1 = rdma(gather_ref.at[FROM_R, q[1]], gather_ref.at[OPP, q[1]],
                      5, 5, left)
        f2r_q0 = rdma(gather_ref.at[FROM_L, q[2]], gather_ref.at[OPP, q[2]],
                      6, 6, right)
        f2r_q1 = rdma(gather_ref.at[FROM_L, q[3]], gather_ref.at[OPP, q[3]],
                      7, 7, right)

        s1_lo_l.wait_recv()
        s1_hi_l.wait_send()
        f2l_q0.start()
        s1_lo_r.wait_recv()
        s1_hi_r.wait_send()
        s1_hi_r.wait_recv()
        f2r_q0.start()

        add_chunk(left, gather_ref.at[FROM_L])
        f2l_q0.wait_send()
        f2l_q1.start()
        s1_hi_l.wait_recv()
        add_chunk(right, gather_ref.at[FROM_R])
        f2r_q0.wait_send()
        f2r_q1.start()

        opp = lax.rem(my + 2, N_DEV)
        f2l_q0.wait_recv()
        add_chunk(opp, gather_ref.at[OPP], 0, e_q)
        f2r_q0.wait_recv()
        add_chunk(opp, gather_ref.at[OPP], 2 * e_q, e_q)
        f2l_q1.wait_recv()
        add_chunk(opp, gather_ref.at[OPP], e_q, e_q)
        f2r_q1.wait_recv()
        add_chunk(opp, gather_ref.at[OPP], 3 * e_q, e_q)

        f2l_q1.wait_send()
        f2r_q1.wait_send()

    return pl.pallas_call(
        body,
        out_shape=jax.ShapeDtypeStruct((n_tok, d_h), jnp.bfloat16),
        in_specs=[
            pl.BlockSpec(memory_space=pltpu.VMEM),
            pl.BlockSpec(memory_space=pltpu.VMEM),
            pl.BlockSpec(memory_space=pltpu.VMEM),
            pl.BlockSpec(memory_space=pltpu.MemorySpace.HBM),
            pl.BlockSpec(memory_space=pltpu.VMEM),
        ],
        out_specs=pl.BlockSpec(memory_space=pltpu.VMEM),
        scratch_shapes=[
            pltpu.VMEM((3, e_per, d_model, d_h), jnp.bfloat16),
            pltpu.VMEM((e_q, d_model, d_h), jnp.float32),
            pltpu.VMEM((e_per, d_model, d_h), jnp.bfloat16),
            pltpu.VMEM((n_tok, 1), jnp.float32),
            pltpu.SemaphoreType.DMA((8,)),
            pltpu.SemaphoreType.DMA((8,)),
            pltpu.SemaphoreType.DMA((2,)),
        ],
        compiler_params=pltpu.CompilerParams(
            collective_id=0, vmem_limit_bytes=52 * 1024 * 1024),
    )(xb, rwb, route_idx, expert_W, swb)


# device time: 172306 ns/iter; 1.0803x vs baseline; 1.0803x over previous
import jax
import jax.numpy as jnp
from jax import lax
from jax.experimental import pallas as pl
from jax.experimental.pallas import tpu as pltpu

N_DEV = 4

FROM_L = 0
FROM_R = 1
OPP = 2


def kernel(x, router_W, route_idx, expert_W, shared_W):
    n_tok, d_model = x.shape
    e_per, _, d_h = expert_W.shape
    e_half = e_per // 2
    e_q = e_half // 2

    xb = x.astype(jnp.bfloat16)
    rwb = router_W.astype(jnp.bfloat16)
    swb = shared_W.astype(jnp.bfloat16)

    def body(x_ref, rw_ref, idx_ref, ew_ref, sw_ref, out_ref,
             gather_ref, stage_ref, own_ref, p_ref, send_sems, recv_sems,
             load_sems):
        my = lax.axis_index("i")
        left = lax.rem(my + N_DEV - 1, N_DEV)
        right = lax.rem(my + 1, N_DEV)

        def rdma(src, dst, si, ri, dev):
            return pltpu.make_async_remote_copy(
                src_ref=src, dst_ref=dst,
                send_sem=send_sems.at[si], recv_sem=recv_sems.at[ri],
                device_id=(dev,), device_id_type=pl.DeviceIdType.MESH,
            )

        def stage_round(r):
            ld = pltpu.make_async_copy(
                ew_ref.at[pl.ds(r * e_q, e_q)], stage_ref,
                load_sems.at[r % 2])
            ld.start()
            ld.wait()
            for k in range(e_q):
                own_ref[r * e_q + k] = stage_ref[k].astype(jnp.bfloat16)

        stage_round(0)

        barrier_sem = pltpu.get_barrier_semaphore()
        for nbr in (left, right):
            pl.semaphore_signal(
                barrier_sem, inc=1,
                device_id=(nbr,), device_id_type=pl.DeviceIdType.MESH,
            )
        pl.semaphore_wait(barrier_sem, 2)
        stage_round(1)

        lo = pl.ds(0, e_half)
        hi = pl.ds(e_half, e_half)
        s1_lo_r = rdma(own_ref.at[lo], gather_ref.at[FROM_L, lo], 0, 0, right)
        s1_lo_l = rdma(own_ref.at[lo], gather_ref.at[FROM_R, lo], 1, 1, left)
        s1_hi_r = rdma(own_ref.at[hi], gather_ref.at[FROM_L, hi], 2, 2, right)
        s1_hi_l = rdma(own_ref.at[hi], gather_ref.at[FROM_R, hi], 3, 3, left)
        s1_lo_r.start()
        s1_lo_l.start()

        stage_round(2)
        stage_round(3)

        idx = idx_ref[...]
        scores = jnp.dot(x_ref[...], rw_ref[...],
                         preferred_element_type=jnp.float32)
        s_max = jnp.max(scores, axis=-1, keepdims=True)
        e_s = jnp.exp(scores - s_max)
        probs = e_s / jnp.sum(e_s, axis=-1, keepdims=True)
        eids = lax.broadcasted_iota(jnp.int32, scores.shape, 1)
        p_ref[...] = jnp.sum(jnp.where(eids == idx, probs, 0.0),
                             axis=-1, keepdims=True)

        s1_lo_r.wait_send()
        s1_hi_r.start()
        s1_lo_l.wait_send()
        s1_hi_l.start()

        TB = 512
        NB = n_tok // TB

        sw = sw_ref[...]

        def shared_blk(b, c):
            sl = pl.ds(b * TB, TB)
            out_ref[sl, :] = jnp.dot(
                x_ref[sl, :], sw, preferred_element_type=jnp.float32
            ).astype(jnp.bfloat16)
            return c

        lax.fori_loop(0, NB, shared_blk, 0)

        def add_chunk(origin, w_ref, j0=0, nj=e_per):
            def blk(b, c):
                sl = pl.ds(b * TB, TB)
                x_blk = x_ref[sl, :]
                idx_blk = idx_ref[sl, :]
                p_blk = p_ref[sl, :]
                acc = out_ref[sl, :].astype(jnp.float32)
                for j in range(j0, j0 + nj):
                    e_glob = origin * e_per + j
                    coeff = jnp.where(idx_blk == e_glob, p_blk,
                                      0.0).astype(jnp.bfloat16)
                    acc = acc + jnp.dot(x_blk * coeff, w_ref[j],
                                        preferred_element_type=jnp.float32)
                out_ref[sl, :] = acc.astype(jnp.bfloat16)
                return c

            lax.fori_loop(0, NB, blk, 0)

        add_chunk(my, own_ref)

        q = [pl.ds(k * e_q, e_q) for k in range(4)]
        f2l_q0 = rdma(gather_ref.at[FROM_R, q[0]], gather_ref.at[OPP, q[0]],
                      4, 4, left)
        f2l_q1 = rdma(gather_ref.at[FROM_R, q[1]], gather_ref.at[OPP, q[1]],
                      5, 5, left)
        f2r_q0 = rdma(gather_ref.at[FROM_L, q[2]], gather_ref.at[OPP, q[2]],
                      6, 6, right)
        f2r_q1 = rdma(gather_ref.at[FROM_L, q[3]], gather_ref.at[OPP, q[3]],
                      7, 7, right)

        s1_lo_r.wait_recv()
        s1_lo_l.wait_recv()
        add_chunk(left, gather_ref.at[FROM_L], 0, e_half)
        add_chunk(right, gather_ref.at[FROM_R], 0, e_half)

        s1_hi_l.wait_send()
        f2l_q0.start()
        s1_hi_r.wait_send()
        s1_hi_r.wait_recv()
        f2r_q0.start()

        s1_hi_l.wait_recv()
        add_chunk(left, gather_ref.at[FROM_L], e_half, e_half)
        add_chunk(right, gather_ref.at[FROM_R], e_half, e_half)

        f2l_q0.wait_send()
        f2l_q1.start()
        f2r_q0.wait_send()
        f2r_q1.start()

        opp = lax.rem(my + 2, N_DEV)
        f2l_q0.wait_recv()
        add_chunk(opp, gather_ref.at[OPP], 0, e_q)
        f2r_q0.wait_recv()
        add_chunk(opp, gather_ref.at[OPP], 2 * e_q, e_q)
        f2l_q1.wait_recv()
        add_chunk(opp, gather_ref.at[OPP], e_q, e_q)
        f2r_q1.wait_recv()
        add_chunk(opp, gather_ref.at[OPP], 3 * e_q, e_q)

        f2l_q1.wait_send()
        f2r_q1.wait_send()

    return pl.pallas_call(
        body,
        out_shape=jax.ShapeDtypeStruct((n_tok, d_h), jnp.bfloat16),
        in_specs=[
            pl.BlockSpec(memory_space=pltpu.VMEM),
            pl.BlockSpec(memory_space=pltpu.VMEM),
            pl.BlockSpec(memory_space=pltpu.VMEM),
            pl.BlockSpec(memory_space=pltpu.MemorySpace.HBM),
            pl.BlockSpec(memory_space=pltpu.VMEM),
        ],
        out_specs=pl.BlockSpec(memory_space=pltpu.VMEM),
        scratch_shapes=[
            pltpu.VMEM((3, e_per, d_model, d_h), jnp.bfloat16),
            pltpu.VMEM((e_q, d_model, d_h), jnp.float32),
            pltpu.VMEM((e_per, d_model, d_h), jnp.bfloat16),
            pltpu.VMEM((n_tok, 1), jnp.float32),
            pltpu.SemaphoreType.DMA((8,)),
            pltpu.SemaphoreType.DMA((8,)),
            pltpu.SemaphoreType.DMA((2,)),
        ],
        compiler_params=pltpu.CompilerParams(
            collective_id=0, vmem_limit_bytes=52 * 1024 * 1024),
    )(xb, rwb, route_idx, expert_W, swb)
